# Initial kernel scaffold; baseline (speedup 1.0000x reference)
import functools

import jax
import jax.numpy as jnp
from jax import lax
from jax.experimental import pallas as pl
from jax.experimental.pallas import tpu as pltpu
from jax.experimental.pallas import tpu_sc as plsc

N_NODES = 10000
N_EDGES = 320000
D_FEAT = 128
N_POOL = 16
POOL_STRIDE = 666

NC = 2
NS = 16
L = 16
NW = NC * NS
EPW = N_EDGES // NW
NV = EPW // L
ACC = N_POOL * D_FEAT


def _sc_body(h_hbm, ei_hbm, part_hbm, srcv, dstv, msrc, mkv, rowv, acc):
    wid = lax.axis_index("s") * NC + lax.axis_index("c")
    base = wid * EPW

    pltpu.sync_copy(ei_hbm.at[0, pl.ds(base, EPW)], srcv)
    pltpu.sync_copy(ei_hbm.at[1, pl.ds(base, EPW)], dstv)

    neg_inf = jnp.full((L,), -jnp.inf, dtype=jnp.float32)

    def init_body(i, carry):
        acc[pl.ds(i * L, L)] = neg_inf
        return carry

    lax.fori_loop(0, ACC // L, init_body, 0)

    stride = jnp.int32(POOL_STRIDE)

    def scan_body(i, n):
        vd = dstv[pl.ds(i * L, L)]
        vs = srcv[pl.ds(i * L, L)]
        q = lax.div(vd, stride)
        mask = (vd - q * stride) == 0
        plsc.store_compressed(msrc.at[pl.ds(n, L)], vs, mask=mask)
        plsc.store_compressed(mkv.at[pl.ds(n, L)], q, mask=mask)
        return n + jnp.sum(mask.astype(jnp.int32))

    n_match = lax.fori_loop(0, NV, scan_body, jnp.int32(0))

    def edge_body(j, carry):
        s = msrc[j]
        k = mkv[j]
        pltpu.sync_copy(h_hbm.at[s], rowv)
        for c in range(D_FEAT // L):
            off = k * D_FEAT + c * L
            acc[pl.ds(off, L)] = jnp.maximum(
                acc[pl.ds(off, L)], rowv[pl.ds(c * L, L)]
            )
        return carry

    lax.fori_loop(0, n_match, edge_body, 0)

    pltpu.sync_copy(acc, part_hbm.at[wid])


def _tc_body(part_ref, out_ref):
    x = part_ref[...]
    m = jnp.max(x, axis=0)
    out_ref[...] = jnp.where(jnp.isneginf(m), 0.0, m)


@jax.jit
def kernel(h, edge_index):
    mesh = plsc.VectorSubcoreMesh(
        core_axis_name="c", subcore_axis_name="s", num_cores=NC,
        num_subcores=NS,
    )
    sc_call = functools.partial(
        pl.kernel,
        out_type=jax.ShapeDtypeStruct((NW, ACC), jnp.float32),
        mesh=mesh,
        scratch_types=[
            pltpu.VMEM((EPW,), jnp.int32),
            pltpu.VMEM((EPW,), jnp.int32),
            pltpu.VMEM((EPW + L,), jnp.int32),
            pltpu.VMEM((EPW + L,), jnp.int32),
            pltpu.VMEM((D_FEAT,), jnp.float32),
            pltpu.VMEM((ACC,), jnp.float32),
        ],
    )(_sc_body)
    part = sc_call(h, edge_index)
    part = part.reshape(NW, N_POOL, D_FEAT)

    out = pl.pallas_call(
        _tc_body,
        out_shape=jax.ShapeDtypeStruct((N_POOL, D_FEAT), jnp.float32),
    )(part)
    return out


# speedup vs baseline: 5.9918x; 5.9918x over previous
import functools

import jax
import jax.numpy as jnp
from jax import lax
from jax.experimental import pallas as pl
from jax.experimental.pallas import tpu as pltpu
from jax.experimental.pallas import tpu_sc as plsc

N_NODES = 10000
N_EDGES = 320000
D_FEAT = 128
N_POOL = 16
POOL_STRIDE = 666

NC = 2
NS = 16
L = 16
NW = NC * NS
EPW = N_EDGES // NW
NV = EPW // L
ACC = N_POOL * D_FEAT
CSTEP = D_FEAT // L


def _sc_body(h_hbm, ei_hbm, part_hbm, srcv, dstv, srcc, dstc, idxb, hrows,
             acc, sem):
    wid = lax.axis_index("s") * NC + lax.axis_index("c")
    base = wid * EPW

    pltpu.sync_copy(ei_hbm.at[pl.ds(base, EPW)], srcv)
    pltpu.sync_copy(ei_hbm.at[pl.ds(N_EDGES + base, EPW)], dstv)

    ninf = jnp.full((L,), -jnp.inf, dtype=jnp.float32)

    def init_body(i, carry):
        acc[pl.ds(i * L, L)] = ninf
        return carry

    lax.fori_loop(0, (ACC + D_FEAT) // L, init_body, 0)

    stride = jnp.int32(POOL_STRIDE)

    def scan_body(i, mc):
        vd = dstv[pl.ds(i * L, L)]
        vs = srcv[pl.ds(i * L, L)]
        q = lax.div(vd, stride)
        mask = (vd - q * stride) == 0
        dstc[pl.ds(mc * L, L)] = vd
        srcc[pl.ds(mc * L, L)] = vs
        cnt = plsc.all_reduce_population_count(mask)[0]
        return mc + ((cnt + (L - 1)) >> 4)

    n_chunks = lax.fori_loop(0, NV, scan_body, jnp.int32(0))

    zero = jnp.full((L,), 0, jnp.int32)
    dump = jnp.full((L,), N_POOL, jnp.int32)

    def chunk_body(t, carry):
        vd = dstc[pl.ds(t * L, L)]
        vs = srcc[pl.ds(t * L, L)]
        q = lax.div(vd, stride)
        mask = (vd - q * stride) == 0
        slot = jnp.where(mask, q, dump)
        idxb[pl.ds(0, L)] = jnp.where(mask, vs, zero)
        pltpu.async_copy(h_hbm.at[idxb], hrows, sem).wait()
        for l in range(L):
            k = slot[l] * D_FEAT
            for c in range(CSTEP):
                acc[pl.ds(k + c * L, L)] = jnp.maximum(
                    acc[pl.ds(k + c * L, L)], hrows[l, pl.ds(c * L, L)]
                )
        return carry

    lax.fori_loop(0, n_chunks, chunk_body, 0)

    pltpu.sync_copy(acc.at[pl.ds(0, ACC)], part_hbm.at[wid])


def _tc_body(part_ref, out_ref):
    x = part_ref[...]
    m = jnp.max(x, axis=0)
    out_ref[...] = jnp.where(jnp.isneginf(m), 0.0, m)


@jax.jit
def kernel(h, edge_index):
    mesh = plsc.VectorSubcoreMesh(
        core_axis_name="c", subcore_axis_name="s", num_cores=NC,
        num_subcores=NS,
    )
    sc_call = functools.partial(
        pl.kernel,
        out_type=jax.ShapeDtypeStruct((NW, ACC), jnp.float32),
        mesh=mesh,
        scratch_types=[
            pltpu.VMEM((EPW,), jnp.int32),
            pltpu.VMEM((EPW,), jnp.int32),
            pltpu.VMEM((EPW + L,), jnp.int32),
            pltpu.VMEM((EPW + L,), jnp.int32),
            pltpu.VMEM((L,), jnp.int32),
            pltpu.VMEM((L, D_FEAT), jnp.float32),
            pltpu.VMEM((ACC + D_FEAT,), jnp.float32),
            pltpu.SemaphoreType.DMA,
        ],
        compiler_params=pltpu.CompilerParams(needs_layout_passes=False),
    )(_sc_body)
    part = sc_call(h, edge_index.reshape(-1))
    part = part.reshape(NW, N_POOL, D_FEAT)

    out = pl.pallas_call(
        _tc_body,
        out_shape=jax.ShapeDtypeStruct((N_POOL, D_FEAT), jnp.float32),
    )(part)
    return out
